# baseline (speedup 1.0000x reference)
import functools

import jax
import jax.numpy as jnp
from jax import lax
from jax.experimental import pallas as pl
from jax.experimental.pallas import tpu as pltpu
from jax.experimental.pallas import tpu_sc as plsc

NUM_NODES = 100000
HDIM = 128
NUM_REL = 16
BATCH = 16384

NUM_CORES = 2
NUM_SUBCORES = 16
LANES = 16
NUM_WORKERS = NUM_CORES * NUM_SUBCORES
B_PER_W = BATCH // NUM_WORKERS
CHUNK = 128
NUM_CHUNKS = B_PER_W // CHUNK
GROUPS = CHUNK // LANES
DSTEP = 4

_mesh = plsc.VectorSubcoreMesh(
    core_axis_name="c",
    subcore_axis_name="s",
    num_cores=NUM_CORES,
    num_subcores=NUM_SUBCORES,
)


@functools.partial(
    pl.kernel,
    out_type=jax.ShapeDtypeStruct((BATCH,), jnp.float32),
    mesh=_mesh,
    scratch_types=[
        pltpu.VMEM((NUM_CHUNKS, CHUNK), jnp.int32),
        pltpu.VMEM((NUM_CHUNKS, CHUNK), jnp.int32),
        pltpu.VMEM((B_PER_W,), jnp.int32),
        pltpu.VMEM((NUM_REL, HDIM), jnp.float32),
        pltpu.VMEM((CHUNK, HDIM), jnp.float32),
        pltpu.VMEM((CHUNK, HDIM), jnp.float32),
        pltpu.VMEM((CHUNK, HDIM), jnp.float32),
        pltpu.VMEM((CHUNK, HDIM), jnp.float32),
        pltpu.VMEM((B_PER_W,), jnp.float32),
        pltpu.SemaphoreType.DMA,
        pltpu.SemaphoreType.DMA,
        pltpu.SemaphoreType.DMA,
        pltpu.SemaphoreType.DMA,
    ],
    compiler_params=pltpu.CompilerParams(needs_layout_passes=False),
)
def _score_kernel(x_hbm, y_hbm, r_hbm, table_hbm, relmat_hbm, out_hbm,
                  xidx_v, yidx_v, ridx_v, rel_v,
                  xe0_v, xe1_v, ye0_v, ye1_v, out_v,
                  sem_x0, sem_x1, sem_y0, sem_y1):
    wid = lax.axis_index("s") * NUM_CORES + lax.axis_index("c")
    base = wid * B_PER_W

    pltpu.sync_copy(r_hbm.at[pl.ds(base, B_PER_W)], ridx_v)
    pltpu.sync_copy(relmat_hbm, rel_v)
    for c in range(NUM_CHUNKS):
        pltpu.sync_copy(x_hbm.at[pl.ds(base + c * CHUNK, CHUNK)], xidx_v.at[c])
        pltpu.sync_copy(y_hbm.at[pl.ds(base + c * CHUNK, CHUNK)], yidx_v.at[c])

    xe_bufs = (xe0_v, xe1_v)
    ye_bufs = (ye0_v, ye1_v)
    sems_x = (sem_x0, sem_x1)
    sems_y = (sem_y0, sem_y1)

    def start_gather(c):
        b = c % 2
        cx = pltpu.async_copy(table_hbm.at[xidx_v.at[c]], xe_bufs[b], sems_x[b])
        cy = pltpu.async_copy(table_hbm.at[yidx_v.at[c]], ye_bufs[b], sems_y[b])
        return cx, cy

    lanes = lax.iota(jnp.int32, LANES)
    zero = jnp.zeros((LANES,), jnp.float32)

    pending = start_gather(0)
    for c in range(NUM_CHUNKS):
        pending[0].wait()
        pending[1].wait()
        if c + 1 < NUM_CHUNKS:
            pending = start_gather(c + 1)
        xe_v = xe_bufs[c % 2]
        ye_v = ye_bufs[c % 2]

        def group_body(g, carry, c=c, xe_v=xe_v, ye_v=ye_v):
            row0 = g * LANES
            rows = lanes + row0
            rvec = ridx_v[pl.ds(c * CHUNK + row0, LANES)]

            @plsc.parallel_loop(0, HDIM, step=DSTEP, unroll=2,
                                carry=(zero, zero, zero, zero))
            def d_body(d, accs):
                upd = []
                for j in range(DSTEP):
                    dvec = (lanes + (d + j)) & (HDIM - 1)
                    xv = plsc.load_gather(xe_v, [rows, dvec])
                    yv = plsc.load_gather(ye_v, [rows, dvec])
                    rv = plsc.load_gather(rel_v, [rvec, dvec])
                    upd.append(xv * yv * rv)
                return tuple(a + u for a, u in zip(accs, upd))

            a0, a1, a2, a3 = d_body
            out_v[pl.ds(c * CHUNK + row0, LANES)] = (a0 + a1) + (a2 + a3)
            return carry

        lax.fori_loop(0, GROUPS, group_body, 0)

    pltpu.sync_copy(out_v, out_hbm.at[pl.ds(base, B_PER_W)])


def kernel(x, y, r, table, R):
    return _score_kernel(
        x.astype(jnp.int32), y.astype(jnp.int32), r.astype(jnp.int32),
        table, R)


# speedup vs baseline: 1.0790x; 1.0790x over previous
import functools

import jax
import jax.numpy as jnp
from jax import lax
from jax.experimental import pallas as pl
from jax.experimental.pallas import tpu as pltpu
from jax.experimental.pallas import tpu_sc as plsc

NUM_NODES = 100000
HDIM = 128
NUM_REL = 16
BATCH = 16384

NUM_CORES = 2
NUM_SUBCORES = 16
LANES = 16
NUM_WORKERS = NUM_CORES * NUM_SUBCORES
B_PER_W = BATCH // NUM_WORKERS
CHUNK = 128
NUM_CHUNKS = B_PER_W // CHUNK
GROUPS = CHUNK // LANES
DSTEP = 4
NBUF = 3

_mesh = plsc.VectorSubcoreMesh(
    core_axis_name="c",
    subcore_axis_name="s",
    num_cores=NUM_CORES,
    num_subcores=NUM_SUBCORES,
)


@functools.partial(
    pl.kernel,
    out_type=jax.ShapeDtypeStruct((BATCH,), jnp.float32),
    mesh=_mesh,
    scratch_types=[
        pltpu.VMEM((B_PER_W,), jnp.int32),
        pltpu.VMEM((B_PER_W,), jnp.int32),
        pltpu.VMEM((B_PER_W,), jnp.int32),
        pltpu.VMEM((NUM_REL, HDIM), jnp.float32),
        pltpu.VMEM((CHUNK, HDIM), jnp.float32),
        pltpu.VMEM((CHUNK, HDIM), jnp.float32),
        pltpu.VMEM((CHUNK, HDIM), jnp.float32),
        pltpu.VMEM((CHUNK, HDIM), jnp.float32),
        pltpu.VMEM((CHUNK, HDIM), jnp.float32),
        pltpu.VMEM((CHUNK, HDIM), jnp.float32),
        pltpu.VMEM((B_PER_W,), jnp.float32),
        pltpu.SemaphoreType.DMA,
        pltpu.SemaphoreType.DMA,
        pltpu.SemaphoreType.DMA,
        pltpu.SemaphoreType.DMA,
        pltpu.SemaphoreType.DMA,
        pltpu.SemaphoreType.DMA,
        pltpu.SemaphoreType.DMA,
        pltpu.SemaphoreType.DMA,
        pltpu.SemaphoreType.DMA,
        pltpu.SemaphoreType.DMA,
    ],
    compiler_params=pltpu.CompilerParams(needs_layout_passes=False),
)
def _score_kernel(x_hbm, y_hbm, r_hbm, table_hbm, relmat_hbm, out_hbm,
                  xidx_v, yidx_v, ridx_v, rel_v,
                  xe0_v, xe1_v, xe2_v, ye0_v, ye1_v, ye2_v, out_v,
                  sem_sx, sem_sy, sem_sr, sem_srel,
                  sem_x0, sem_x1, sem_x2, sem_y0, sem_y1, sem_y2):
    wid = lax.axis_index("s") * NUM_CORES + lax.axis_index("c")
    base = wid * B_PER_W

    st_x = pltpu.async_copy(x_hbm.at[pl.ds(base, B_PER_W)], xidx_v, sem_sx)
    st_y = pltpu.async_copy(y_hbm.at[pl.ds(base, B_PER_W)], yidx_v, sem_sy)
    st_r = pltpu.async_copy(r_hbm.at[pl.ds(base, B_PER_W)], ridx_v, sem_sr)
    st_rel = pltpu.async_copy(relmat_hbm, rel_v, sem_srel)

    xe_bufs = (xe0_v, xe1_v, xe2_v)
    ye_bufs = (ye0_v, ye1_v, ye2_v)
    sems_x = (sem_x0, sem_x1, sem_x2)
    sems_y = (sem_y0, sem_y1, sem_y2)

    def start_gather(c):
        b = c % NBUF
        cx = pltpu.async_copy(
            table_hbm.at[xidx_v.at[pl.ds(c * CHUNK, CHUNK)]],
            xe_bufs[b], sems_x[b])
        cy = pltpu.async_copy(
            table_hbm.at[yidx_v.at[pl.ds(c * CHUNK, CHUNK)]],
            ye_bufs[b], sems_y[b])
        return cx, cy

    st_x.wait()
    st_y.wait()
    pending = [start_gather(c) for c in range(NBUF)]
    st_r.wait()
    st_rel.wait()

    lanes = lax.iota(jnp.int32, LANES)
    zero = jnp.zeros((LANES,), jnp.float32)

    for c in range(NUM_CHUNKS):
        cx, cy = pending[c % NBUF]
        cx.wait()
        cy.wait()
        xe_v = xe_bufs[c % NBUF]
        ye_v = ye_bufs[c % NBUF]

        def group_body(g, carry, c=c, xe_v=xe_v, ye_v=ye_v):
            row0 = g * LANES
            rows = lanes + row0
            rvec = ridx_v[pl.ds(c * CHUNK + row0, LANES)]

            @plsc.parallel_loop(0, HDIM, step=DSTEP, unroll=2,
                                carry=(zero, zero, zero, zero))
            def d_body(d, accs):
                upd = []
                for j in range(DSTEP):
                    dvec = (lanes + (d + j)) & (HDIM - 1)
                    xv = plsc.load_gather(xe_v, [rows, dvec])
                    yv = plsc.load_gather(ye_v, [rows, dvec])
                    rv = plsc.load_gather(rel_v, [rvec, dvec])
                    upd.append(xv * yv * rv)
                return tuple(a + u for a, u in zip(accs, upd))

            a0, a1, a2, a3 = d_body
            out_v[pl.ds(c * CHUNK + row0, LANES)] = (a0 + a1) + (a2 + a3)
            return carry

        lax.fori_loop(0, GROUPS, group_body, 0)
        if c + NBUF < NUM_CHUNKS:
            pending[c % NBUF] = start_gather(c + NBUF)

    pltpu.sync_copy(out_v, out_hbm.at[pl.ds(base, B_PER_W)])


def kernel(x, y, r, table, R):
    return _score_kernel(
        x.astype(jnp.int32), y.astype(jnp.int32), r.astype(jnp.int32),
        table, R)


# speedup vs baseline: 1.1166x; 1.0349x over previous
import functools

import jax
import jax.numpy as jnp
from jax import lax
from jax.experimental import pallas as pl
from jax.experimental.pallas import tpu as pltpu
from jax.experimental.pallas import tpu_sc as plsc

NUM_NODES = 100000
HDIM = 128
NUM_REL = 16
BATCH = 16384

NUM_CORES = 2
NUM_SUBCORES = 16
LANES = 16
NUM_WORKERS = NUM_CORES * NUM_SUBCORES
B_PER_W = BATCH // NUM_WORKERS
CHUNK = 128
NUM_CHUNKS = B_PER_W // CHUNK
GROUPS = CHUNK // LANES
DSTEP = 4

_mesh = plsc.VectorSubcoreMesh(
    core_axis_name="c",
    subcore_axis_name="s",
    num_cores=NUM_CORES,
    num_subcores=NUM_SUBCORES,
)


@functools.partial(
    pl.kernel,
    out_type=jax.ShapeDtypeStruct((BATCH,), jnp.float32),
    mesh=_mesh,
    scratch_types=[
        pltpu.VMEM((NUM_CHUNKS, CHUNK), jnp.int32),
        pltpu.VMEM((NUM_CHUNKS, CHUNK), jnp.int32),
        pltpu.VMEM((B_PER_W,), jnp.int32),
        pltpu.VMEM((NUM_REL, HDIM), jnp.float32),
        pltpu.VMEM((2, CHUNK, HDIM), jnp.float32),
        pltpu.VMEM((2, CHUNK, HDIM), jnp.float32),
        pltpu.VMEM((B_PER_W,), jnp.float32),
        pltpu.SemaphoreType.DMA,
        pltpu.SemaphoreType.DMA,
        pltpu.SemaphoreType.DMA,
        pltpu.SemaphoreType.DMA,
        pltpu.SemaphoreType.DMA,
        pltpu.SemaphoreType.DMA,
        pltpu.SemaphoreType.DMA,
        pltpu.SemaphoreType.DMA,
    ],
    compiler_params=pltpu.CompilerParams(needs_layout_passes=False),
)
def _score_kernel(x_hbm, y_hbm, r_hbm, table_hbm, relmat_hbm, out_hbm,
                  xidx_v, yidx_v, ridx_v, rel_v, xe_v, ye_v, out_v,
                  sem_sx, sem_sy, sem_sr, sem_srel,
                  sem_x0, sem_x1, sem_y0, sem_y1):
    wid = lax.axis_index("s") * NUM_CORES + lax.axis_index("c")
    base = wid * B_PER_W

    st_x = pltpu.async_copy(
        x_hbm.at[pl.ds(wid * NUM_CHUNKS, NUM_CHUNKS)], xidx_v, sem_sx)
    st_y = pltpu.async_copy(
        y_hbm.at[pl.ds(wid * NUM_CHUNKS, NUM_CHUNKS)], yidx_v, sem_sy)
    st_r = pltpu.async_copy(r_hbm.at[pl.ds(base, B_PER_W)], ridx_v, sem_sr)
    st_rel = pltpu.async_copy(relmat_hbm, rel_v, sem_srel)

    sems_x = (sem_x0, sem_x1)
    sems_y = (sem_y0, sem_y1)

    st_x.wait()
    st_y.wait()
    for c in range(2):
        pltpu.async_copy(table_hbm.at[xidx_v.at[c]], xe_v.at[c], sems_x[c])
        pltpu.async_copy(table_hbm.at[yidx_v.at[c]], ye_v.at[c], sems_y[c])
    st_r.wait()
    st_rel.wait()

    lanes = lax.iota(jnp.int32, LANES)
    zero = jnp.zeros((LANES,), jnp.float32)

    def chunk_body(c, carry):
        par = lax.rem(c, 2)

        def wait_bufs(b):
            pltpu.make_async_copy(
                table_hbm.at[xidx_v.at[b]], xe_v.at[b], sems_x[b]).wait()
            pltpu.make_async_copy(
                table_hbm.at[yidx_v.at[b]], ye_v.at[b], sems_y[b]).wait()

        pl.when(par == 0)(lambda: wait_bufs(0))
        pl.when(par == 1)(lambda: wait_bufs(1))

        bsplat = jnp.full((LANES,), par, jnp.int32)

        def group_body(g, carry2):
            row0 = g * LANES
            rows = lanes + row0
            rvec = ridx_v[pl.ds(c * CHUNK + row0, LANES)]

            @plsc.parallel_loop(0, HDIM, step=DSTEP, unroll=2,
                                carry=(zero, zero, zero, zero))
            def d_body(d, accs):
                upd = []
                for j in range(DSTEP):
                    dvec = (lanes + (d + j)) & (HDIM - 1)
                    xv = plsc.load_gather(xe_v, [bsplat, rows, dvec])
                    yv = plsc.load_gather(ye_v, [bsplat, rows, dvec])
                    rv = plsc.load_gather(rel_v, [rvec, dvec])
                    upd.append(xv * yv * rv)
                return tuple(a + u for a, u in zip(accs, upd))

            a0, a1, a2, a3 = d_body
            out_v[pl.ds(c * CHUNK + row0, LANES)] = (a0 + a1) + (a2 + a3)
            return carry2

        lax.fori_loop(0, GROUPS, group_body, 0)

        def start_next(b):
            nxt = c + 2
            pltpu.async_copy(
                table_hbm.at[xidx_v.at[nxt]], xe_v.at[b], sems_x[b])
            pltpu.async_copy(
                table_hbm.at[yidx_v.at[nxt]], ye_v.at[b], sems_y[b])

        pl.when((par == 0) & (c + 2 < NUM_CHUNKS))(lambda: start_next(0))
        pl.when((par == 1) & (c + 2 < NUM_CHUNKS))(lambda: start_next(1))
        return carry

    lax.fori_loop(0, NUM_CHUNKS, chunk_body, 0)

    pltpu.sync_copy(out_v, out_hbm.at[pl.ds(base, B_PER_W)])


def kernel(x, y, r, table, R):
    x2 = x.astype(jnp.int32).reshape(NUM_WORKERS * NUM_CHUNKS, CHUNK)
    y2 = y.astype(jnp.int32).reshape(NUM_WORKERS * NUM_CHUNKS, CHUNK)
    return _score_kernel(x2, y2, r.astype(jnp.int32), table, R)
